# baseline (speedup 1.0000x reference)
import functools

import jax
import jax.numpy as jnp
from jax.experimental import pallas as pl
from jax.experimental.pallas import tpu as pltpu

H = 1024
FF = 2048
E = 8
TOP_K = 2
T = 512

KF = 4
FC = FF // KF
R = T * TOP_K
MB = 128
NMB = R // MB
NW = NMB + E - 1

_NEG = -3.0e38


def _router_shared_kernel(x_ref, rw_ref, wg_ref, wu_ref, wd_ref,
                          shared_ref, tw_ref, ti_ref):
    kf = pl.program_id(0)
    x = x_ref[...]

    @pl.when(kf == 0)
    def _router():
        logits = jax.lax.dot_general(
            x, rw_ref[...], (((1,), (1,)), ((), ())),
            preferred_element_type=jnp.float32)
        ii = jax.lax.broadcasted_iota(jnp.int32, (T, E), 1)
        m1 = jnp.max(logits, axis=1, keepdims=True)
        i1 = jnp.min(jnp.where(logits == m1, ii, E), axis=1, keepdims=True)
        l2 = jnp.where(ii == i1, _NEG, logits)
        m2 = jnp.max(l2, axis=1, keepdims=True)
        i2 = jnp.min(jnp.where(l2 == m2, ii, E), axis=1, keepdims=True)
        w1 = jax.nn.sigmoid(m1 - m2)
        w2 = jax.nn.sigmoid(m2 - m1)
        tw_ref[...] = jnp.concatenate([w1, w2], axis=1)
        ti_ref[...] = jnp.concatenate([i1, i2], axis=1)

    gate = jax.lax.dot_general(x, wg_ref[...], (((1,), (1,)), ((), ())),
                               preferred_element_type=jnp.float32)
    up = jax.lax.dot_general(x, wu_ref[...], (((1,), (1,)), ((), ())),
                             preferred_element_type=jnp.float32)
    h = gate * jax.nn.sigmoid(gate) * up
    contrib = jax.lax.dot_general(h, wd_ref[...], (((1,), (1,)), ((), ())),
                                  preferred_element_type=jnp.float32)

    @pl.when(kf == 0)
    def _init():
        shared_ref[...] = contrib

    @pl.when(kf != 0)
    def _acc():
        shared_ref[...] += contrib


def _grouped_kernel(ie_ref, imb_ref, ifirst_ref, ivalid_ref,
                    x_ref, eg_ref, eu_ref, ed_ref, ts_ref, es_ref,
                    ys_ref, xb_ref):
    i = pl.program_id(0)
    kf = pl.program_id(1)

    @pl.when(kf == 0)
    def _gather():
        e_i = ie_ref[i]
        valid = ivalid_ref[i] > 0
        row_t = ts_ref[...]
        row_e = es_ref[...]
        tok = jax.lax.broadcasted_iota(jnp.int32, (MB, T), 1)
        g = (row_t == tok) & (row_e == e_i) & valid
        xb_ref[...] = jnp.dot(g.astype(jnp.float32), x_ref[...],
                              preferred_element_type=jnp.float32)

    xb = xb_ref[...]
    gate = jax.lax.dot_general(xb, eg_ref[0], (((1,), (1,)), ((), ())),
                               preferred_element_type=jnp.float32)
    up = jax.lax.dot_general(xb, eu_ref[0], (((1,), (1,)), ((), ())),
                             preferred_element_type=jnp.float32)
    h = gate * jax.nn.sigmoid(gate) * up
    contrib = jax.lax.dot_general(h, ed_ref[0], (((1,), (1,)), ((), ())),
                                  preferred_element_type=jnp.float32)

    first = (ifirst_ref[i] > 0) & (kf == 0)

    @pl.when(first)
    def _init():
        ys_ref[...] = contrib

    @pl.when(jnp.logical_not(first))
    def _acc():
        ys_ref[...] += contrib


def _combine_kernel(ys_ref, shared_ref, tr_ref, wr_ref, out_ref):
    ti = jax.lax.broadcasted_iota(jnp.int32, (T, R), 0)
    c = jnp.where(tr_ref[...] == ti, wr_ref[...], jnp.float32(0.0))
    out_ref[...] = shared_ref[...] + jnp.dot(
        c, ys_ref[...], preferred_element_type=jnp.float32)


@jax.jit
def kernel(x, router_weight, w_gate, w_up, w_down,
           expert_gate, expert_up, expert_down):
    shared_out, top_w, top_i = pl.pallas_call(
        _router_shared_kernel,
        grid=(KF,),
        in_specs=[
            pl.BlockSpec((T, H), lambda kf: (0, 0)),
            pl.BlockSpec((E, H), lambda kf: (0, 0)),
            pl.BlockSpec((FC, H), lambda kf: (kf, 0)),
            pl.BlockSpec((FC, H), lambda kf: (kf, 0)),
            pl.BlockSpec((H, FC), lambda kf: (0, kf)),
        ],
        out_specs=[
            pl.BlockSpec((T, H), lambda kf: (0, 0)),
            pl.BlockSpec((T, TOP_K), lambda kf: (0, 0)),
            pl.BlockSpec((T, TOP_K), lambda kf: (0, 0)),
        ],
        out_shape=[
            jax.ShapeDtypeStruct((T, H), jnp.float32),
            jax.ShapeDtypeStruct((T, TOP_K), jnp.float32),
            jax.ShapeDtypeStruct((T, TOP_K), jnp.int32),
        ],
    )(x, router_weight, w_gate, w_up, w_down)

    e_flat = top_i.reshape(-1)
    w_flat = top_w.reshape(-1)
    perm = jnp.argsort(e_flat, stable=True).astype(jnp.int32)
    e_sorted = e_flat[perm]
    t_sorted = perm // jnp.int32(TOP_K)
    w_sorted = w_flat[perm]

    mbv = jnp.arange(R, dtype=jnp.int32) // MB
    pair = mbv * E + e_sorted
    firstflag = jnp.concatenate(
        [jnp.ones((1,), jnp.bool_), pair[1:] != pair[:-1]])
    n_items = jnp.sum(firstflag.astype(jnp.int32))
    pos = jnp.nonzero(firstflag, size=NW, fill_value=R - 1)[0].astype(jnp.int32)
    item_e = e_sorted[pos]
    item_mb = pos // MB
    item_first = jnp.concatenate(
        [jnp.ones((1,), jnp.int32),
         (item_mb[1:] != item_mb[:-1]).astype(jnp.int32)])
    item_valid = (jnp.arange(NW, dtype=jnp.int32) < n_items).astype(jnp.int32)

    ys = pl.pallas_call(
        _grouped_kernel,
        grid_spec=pltpu.PrefetchScalarGridSpec(
            num_scalar_prefetch=4,
            grid=(NW, KF),
            in_specs=[
                pl.BlockSpec((T, H), lambda i, kf, ie, imb, ifi, iva: (0, 0)),
                pl.BlockSpec((1, FC, H),
                             lambda i, kf, ie, imb, ifi, iva: (ie[i], kf, 0)),
                pl.BlockSpec((1, FC, H),
                             lambda i, kf, ie, imb, ifi, iva: (ie[i], kf, 0)),
                pl.BlockSpec((1, H, FC),
                             lambda i, kf, ie, imb, ifi, iva: (ie[i], 0, kf)),
                pl.BlockSpec((MB, 1),
                             lambda i, kf, ie, imb, ifi, iva: (imb[i], 0)),
                pl.BlockSpec((MB, 1),
                             lambda i, kf, ie, imb, ifi, iva: (imb[i], 0)),
            ],
            out_specs=pl.BlockSpec(
                (MB, H), lambda i, kf, ie, imb, ifi, iva: (imb[i], 0)),
            scratch_shapes=[pltpu.VMEM((MB, H), jnp.float32)],
        ),
        out_shape=jax.ShapeDtypeStruct((R, H), jnp.float32),
    )(item_e, item_mb, item_first, item_valid,
      x, expert_gate, expert_up, expert_down,
      t_sorted.reshape(R, 1), e_sorted.reshape(R, 1))

    out = pl.pallas_call(
        _combine_kernel,
        in_specs=[
            pl.BlockSpec((R, H), lambda: (0, 0)),
            pl.BlockSpec((T, H), lambda: (0, 0)),
            pl.BlockSpec((1, R), lambda: (0, 0)),
            pl.BlockSpec((1, R), lambda: (0, 0)),
        ],
        out_specs=pl.BlockSpec((T, H), lambda: (0, 0)),
        out_shape=jax.ShapeDtypeStruct((T, H), jnp.float32),
    )(ys, shared_out, t_sorted.reshape(1, R), w_sorted.reshape(1, R))
    return out
